# baseline (speedup 1.0000x reference)
import functools

import jax
import jax.numpy as jnp
from jax import lax
from jax.experimental import pallas as pl
from jax.experimental.pallas import tpu as pltpu
from jax.experimental.pallas import tpu_sc as plsc

N = 10000
E = 320000
D = 128
H = 256
HH = H // 2
G = 128
C = 16

NC = 2
NS = 16
CHUNK = 128
NCH = 160
EP = NCH * NS * CHUNK
EPT = EP // NS
EPW = EP // (NC * NS)
DCH = EPW // CHUNK
NJUNK = 16
NPAD = 10112
RPT = NPAD // NS


def _deg_body(dst2_hbm, ones_hbm, zero_hbm, out0, out1, didx2, ones_v, acc,
              sem):
    c = lax.axis_index("c")
    s = lax.axis_index("s")
    pltpu.sync_copy(zero_hbm, acc.at[pl.ds(s * RPT, RPT)])
    pltpu.sync_copy(ones_hbm, ones_v)
    w = c * NS + s
    pltpu.sync_copy(dst2_hbm.at[pl.ds(w * DCH, DCH)], didx2)
    plsc.subcore_barrier()

    def body(j, carry):
        pltpu.sync_copy(ones_v, acc.at[didx2.at[j]], add=True)
        return carry

    lax.fori_loop(0, DCH, body, 0)
    plsc.subcore_barrier()

    @pl.when(c == 0)
    def _():
        pltpu.sync_copy(acc.at[pl.ds(s * RPT, RPT)], out0.at[pl.ds(s * RPT, RPT)])

    @pl.when(c == 1)
    def _():
        pltpu.sync_copy(acc.at[pl.ds(s * RPT, RPT)], out1.at[pl.ds(s * RPT, RPT)])


def _scatter_body(u0_hbm, u1_hbm, src2_hbm, dst2_hbm, zero_hbm,
                  out0, out1, sidx, didx, rows, acc,
                  gsem0, gsem1, isem0, isem1, ssem0, ssem1):
    c = lax.axis_index("c")
    s = lax.axis_index("s")
    gsems = (gsem0, gsem1)
    isems = (isem0, isem1)
    ssems = (ssem0, ssem1)
    pltpu.sync_copy(zero_hbm, acc.at[pl.ds(s * RPT, RPT)])
    plsc.subcore_barrier()

    def idxload(j, b):
        pltpu.async_copy(src2_hbm.at[s * NCH + j], sidx.at[b], isems[b])
        pltpu.async_copy(dst2_hbm.at[s * NCH + j], didx.at[b], isems[b])

    def idxwait(b):
        pltpu.make_async_copy(src2_hbm.at[0], sidx.at[b], isems[b]).wait()
        pltpu.make_async_copy(dst2_hbm.at[0], didx.at[b], isems[b]).wait()

    def gather(b):
        @pl.when(c == 0)
        def _():
            pltpu.async_copy(u0_hbm.at[sidx.at[b]], rows.at[b], gsems[b])

        @pl.when(c == 1)
        def _():
            pltpu.async_copy(u1_hbm.at[sidx.at[b]], rows.at[b], gsems[b])

    def gwait(b):
        pltpu.make_async_copy(u0_hbm.at[sidx.at[b]], rows.at[b],
                              gsems[b]).wait()

    def sstart(b):
        pltpu.make_async_copy(rows.at[b], acc.at[didx.at[b]],
                              ssems[b]).start(add=True)

    def swait(b):
        pltpu.make_async_copy(rows.at[b], acc.at[didx.at[b]],
                              ssems[b]).wait()

    idxload(0, 0)
    idxwait(0)
    gather(0)

    def body(j2, carry):
        for b in range(2):
            j = j2 * 2 + b
            gwait(b)
            sstart(b)

            @pl.when(j >= 1)
            def _():
                swait(1 - b)

            @pl.when(j + 1 < NCH)
            def _():
                idxload(j + 1, 1 - b)
                idxwait(1 - b)
                gather(1 - b)
        return carry

    lax.fori_loop(0, NCH // 2, body, 0)
    swait((NCH - 1) % 2)
    plsc.subcore_barrier()

    @pl.when(c == 0)
    def _():
        pltpu.sync_copy(acc.at[pl.ds(s * RPT, RPT)], out0.at[pl.ds(s * RPT, RPT)])

    @pl.when(c == 1)
    def _():
        pltpu.sync_copy(acc.at[pl.ds(s * RPT, RPT)], out1.at[pl.ds(s * RPT, RPT)])


@functools.lru_cache(maxsize=None)
def _sc_kernels():
    mesh = plsc.VectorSubcoreMesh(
        core_axis_name="c", subcore_axis_name="s",
        num_cores=NC, num_subcores=NS)
    deg_k = pl.kernel(
        _deg_body,
        out_type=(jax.ShapeDtypeStruct((NPAD, HH), jnp.float32),
                  jax.ShapeDtypeStruct((NPAD, HH), jnp.float32)),
        mesh=mesh,
        scratch_types=[
            pltpu.VMEM((DCH, CHUNK), jnp.int32),
            pltpu.VMEM((CHUNK, HH), jnp.float32),
            pltpu.VMEM_SHARED((NPAD, HH), jnp.float32),
            pltpu.SemaphoreType.DMA,
        ],
    )
    scat_k = pl.kernel(
        _scatter_body,
        out_type=(jax.ShapeDtypeStruct((NPAD, HH), jnp.float32),
                  jax.ShapeDtypeStruct((NPAD, HH), jnp.float32)),
        mesh=mesh,
        scratch_types=[
            pltpu.VMEM((2, CHUNK), jnp.int32),
            pltpu.VMEM((2, CHUNK), jnp.int32),
            pltpu.VMEM((2, CHUNK, HH), jnp.float32),
            pltpu.VMEM_SHARED((NPAD, HH), jnp.float32),
            pltpu.SemaphoreType.DMA,
            pltpu.SemaphoreType.DMA,
            pltpu.SemaphoreType.DMA,
            pltpu.SemaphoreType.DMA,
            pltpu.SemaphoreType.DMA,
            pltpu.SemaphoreType.DMA,
        ],
    )
    return deg_k, scat_k


def _mm0_body(x_ref, w_ref, o_ref):
    o_ref[...] = jnp.dot(x_ref[...], w_ref[...],
                         preferred_element_type=jnp.float32)


def _scale_body(deg0_ref, deg1_ref, hw_ref, o0_ref, o1_ref, dinv_ref):
    deg = deg0_ref[...][:N, 0:1] + deg1_ref[...][:N, 0:1]
    dinv = lax.rsqrt(deg + 1.0)
    u = hw_ref[...] * dinv
    o0_ref[...] = u[:, :HH]
    o1_ref[...] = u[:, HH:]
    dinv_ref[...] = jnp.broadcast_to(dinv, (N, 8))


def _epi_body(dinv_ref, a0_ref, a1_ref, u0_ref, u1_ref, b_ref, g_ref, be_ref,
              w_ref, o0_ref, o1_ref):
    dinv = dinv_ref[...][:, 0:1]
    acc = jnp.concatenate([a0_ref[...][:N], a1_ref[...][:N]], axis=1)
    u = jnp.concatenate([u0_ref[...], u1_ref[...]], axis=1)
    conv = dinv * (acc + u) + b_ref[...]
    h = jnp.maximum(conv, 0.0)
    m = jnp.mean(h, axis=0, keepdims=True)
    v = jnp.mean((h - m) * (h - m), axis=0, keepdims=True)
    hbn = g_ref[...] * (h - m) * lax.rsqrt(v + 1e-5) + be_ref[...]
    un = jnp.dot(hbn, w_ref[...], preferred_element_type=jnp.float32) * dinv
    o0_ref[...] = un[:, :HH]
    o1_ref[...] = un[:, HH:]


def _head_body(dinv_ref, a0_ref, a1_ref, u0_ref, u1_ref, b_ref, batch_ref,
               y_ref, muw_ref, mub_ref, sgw_ref, sgb_ref, f1w_ref, f1b_ref,
               f2w_ref, f2b_ref, logits_ref, loss_ref):
    dinv = dinv_ref[...][:, 0:1]
    acc = jnp.concatenate([a0_ref[...][:N], a1_ref[...][:N]], axis=1)
    u = jnp.concatenate([u0_ref[...], u1_ref[...]], axis=1)
    h = jnp.maximum(dinv * (acc + u) + b_ref[...], 0.0)
    gids = lax.broadcasted_iota(jnp.int32, (1, G), 1)
    p = (batch_ref[...] == gids).astype(jnp.float32)
    psum = lax.dot_general(p, h, (((0,), (0,)), ((), ())),
                           preferred_element_type=jnp.float32)
    ones = jnp.full((N, 1), 1.0, jnp.float32)
    cnt = lax.dot_general(p, ones, (((0,), (0,)), ((), ())),
                          preferred_element_type=jnp.float32)
    pooled = psum / jnp.maximum(cnt, 1.0)
    mu = jnp.dot(pooled, muw_ref[...], preferred_element_type=jnp.float32) \
        + mub_ref[...]
    sp = jnp.dot(pooled, sgw_ref[...], preferred_element_type=jnp.float32) \
        + sgb_ref[...]
    sigma = jnp.maximum(sp, 0.0) + jnp.log(1.0 + jnp.exp(-jnp.abs(sp)))
    ms = jnp.concatenate([mu, sigma], axis=1)
    hid = jnp.maximum(
        jnp.dot(ms, f1w_ref[...], preferred_element_type=jnp.float32)
        + f1b_ref[...], 0.0)
    logits = jnp.dot(hid, f2w_ref[...], preferred_element_type=jnp.float32) \
        + f2b_ref[...]
    mx = jnp.max(logits, axis=1, keepdims=True)
    sh = logits - mx
    lse = jnp.log(jnp.sum(jnp.exp(sh), axis=1, keepdims=True))
    logp = sh - lse
    cids = lax.broadcasted_iota(jnp.int32, (1, C), 1)
    picked = jnp.where(y_ref[...] == cids, logp, 0.0)
    loss = -jnp.sum(picked) / G
    logits_ref[...] = logits
    loss_ref[...] = jnp.reshape(loss, (1, 1))


def _tc_call(body, out_shapes):
    return pl.pallas_call(body, out_shape=out_shapes)


def kernel(x, edge_index, batch, y, W0, b0, g0, be0, W1, b1, g1, be1, W2, b2,
           muW, mub, sgW, sgb, f1W, f1b, f2W, f2b):
    src = edge_index[0].astype(jnp.int32)
    dst = edge_index[1].astype(jnp.int32)
    npad = EP - E
    pad_i = jnp.arange(npad, dtype=jnp.int32)
    src_p = jnp.concatenate([src, (pad_i * 37) % N]).reshape(EP // CHUNK, CHUNK)
    dst_p = jnp.concatenate([dst, N + (pad_i % NJUNK)]).reshape(
        EP // CHUNK, CHUNK)

    ones_h = jnp.ones((CHUNK, HH), jnp.float32)
    zero_h = jnp.zeros((RPT, HH), jnp.float32)

    deg_kernel, scatter_kernel = _sc_kernels()
    deg0, deg1 = deg_kernel(dst_p, ones_h, zero_h)

    hw0 = _tc_call(_mm0_body, jax.ShapeDtypeStruct((N, H), jnp.float32))(x, W0)
    uo = jax.ShapeDtypeStruct((N, HH), jnp.float32)
    dvo = jax.ShapeDtypeStruct((N, 8), jnp.float32)
    u0_lo, u0_hi, dinv = _tc_call(_scale_body, (uo, uo, dvo))(deg0, deg1, hw0)
    u0 = (u0_lo, u0_hi)

    a0 = scatter_kernel(u0[0], u0[1], src_p, dst_p, zero_h)
    u1 = _tc_call(_epi_body, (uo, uo))(
        dinv, a0[0], a0[1], u0[0], u0[1], b0.reshape(1, H), g0.reshape(1, H),
        be0.reshape(1, H), W1)

    a1 = scatter_kernel(u1[0], u1[1], src_p, dst_p, zero_h)
    u2 = _tc_call(_epi_body, (uo, uo))(
        dinv, a1[0], a1[1], u1[0], u1[1], b1.reshape(1, H), g1.reshape(1, H),
        be1.reshape(1, H), W2)

    a2 = scatter_kernel(u2[0], u2[1], src_p, dst_p, zero_h)
    logits, loss = _tc_call(
        _head_body, (jax.ShapeDtypeStruct((G, C), jnp.float32),
                     jax.ShapeDtypeStruct((1, 1), jnp.float32)))(
        dinv, a2[0], a2[1], u2[0], u2[1], b2.reshape(1, H),
        batch.astype(jnp.int32).reshape(N, 1), y.astype(jnp.int32).reshape(G, 1),
        muW, mub.reshape(1, H), sgW, sgb.reshape(1, H), f1W, f1b.reshape(1, H),
        f2W, f2b.reshape(1, C))
    return logits, loss.reshape(())


# speedup vs baseline: 1.2888x; 1.2888x over previous
import functools

import jax
import jax.numpy as jnp
from jax import lax
from jax.experimental import pallas as pl
from jax.experimental.pallas import tpu as pltpu
from jax.experimental.pallas import tpu_sc as plsc

N = 10000
E = 320000
D = 128
H = 256
HH = H // 2
G = 128
C = 16

NC = 2
NS = 16
CHUNK = 128
NCH = 160
EP = NCH * NS * CHUNK
EPT = EP // NS
EPW = EP // (NC * NS)
DCH = EPW // CHUNK
NJUNK = 16
NPAD = 10112
RPT = NPAD // NS


def _deg_body(dst2_hbm, ones_hbm, zero_hbm, out0, out1, didx2, ones_v, acc,
              sem):
    c = lax.axis_index("c")
    s = lax.axis_index("s")
    pltpu.sync_copy(zero_hbm, acc.at[pl.ds(s * RPT, RPT)])
    pltpu.sync_copy(ones_hbm, ones_v)
    w = c * NS + s
    pltpu.sync_copy(dst2_hbm.at[pl.ds(w * DCH, DCH)], didx2)
    plsc.subcore_barrier()

    def body(j, carry):
        pltpu.sync_copy(ones_v, acc.at[didx2.at[j]], add=True)
        return carry

    lax.fori_loop(0, DCH, body, 0)
    plsc.subcore_barrier()

    @pl.when(c == 0)
    def _():
        pltpu.sync_copy(acc.at[pl.ds(s * RPT, RPT)], out0.at[pl.ds(s * RPT, RPT)])

    @pl.when(c == 1)
    def _():
        pltpu.sync_copy(acc.at[pl.ds(s * RPT, RPT)], out1.at[pl.ds(s * RPT, RPT)])


def _scatter_body(u0_hbm, u1_hbm, src2_hbm, dst2_hbm, zero_hbm,
                  out0, out1, sidx, didx, rows, acc,
                  gsem0, gsem1, isem0, isem1, ssem0, ssem1):
    c = lax.axis_index("c")
    s = lax.axis_index("s")
    gsems = (gsem0, gsem1)
    isems = (isem0, isem1)
    ssems = (ssem0, ssem1)
    pltpu.sync_copy(zero_hbm, acc.at[pl.ds(s * RPT, RPT)])
    plsc.subcore_barrier()

    def idxload(j, b):
        pltpu.async_copy(src2_hbm.at[s * NCH + j], sidx.at[b], isems[b])
        pltpu.async_copy(dst2_hbm.at[s * NCH + j], didx.at[b], isems[b])

    def idxwait(b):
        pltpu.make_async_copy(src2_hbm.at[0], sidx.at[b], isems[b]).wait()
        pltpu.make_async_copy(dst2_hbm.at[0], didx.at[b], isems[b]).wait()

    def gather(b):
        @pl.when(c == 0)
        def _():
            pltpu.async_copy(u0_hbm.at[sidx.at[b]], rows.at[b], gsems[b])

        @pl.when(c == 1)
        def _():
            pltpu.async_copy(u1_hbm.at[sidx.at[b]], rows.at[b], gsems[b])

    def gwait(b):
        pltpu.make_async_copy(u0_hbm.at[sidx.at[b]], rows.at[b],
                              gsems[b]).wait()

    idxload(0, 0)
    idxload(1, 1)
    idxwait(0)
    gather(0)

    def body(j2, carry):
        for b in range(2):
            j = j2 * 2 + b

            @pl.when(j + 1 < NCH)
            def _():
                idxwait(1 - b)
                gather(1 - b)

            gwait(b)
            pltpu.sync_copy(rows.at[b], acc.at[didx.at[b]], add=True)

            @pl.when(j + 2 < NCH)
            def _():
                idxload(j + 2, b)
        return carry

    lax.fori_loop(0, NCH // 2, body, 0)
    plsc.subcore_barrier()

    @pl.when(c == 0)
    def _():
        pltpu.sync_copy(acc.at[pl.ds(s * RPT, RPT)], out0.at[pl.ds(s * RPT, RPT)])

    @pl.when(c == 1)
    def _():
        pltpu.sync_copy(acc.at[pl.ds(s * RPT, RPT)], out1.at[pl.ds(s * RPT, RPT)])


@functools.lru_cache(maxsize=None)
def _sc_kernels():
    mesh = plsc.VectorSubcoreMesh(
        core_axis_name="c", subcore_axis_name="s",
        num_cores=NC, num_subcores=NS)
    deg_k = pl.kernel(
        _deg_body,
        out_type=(jax.ShapeDtypeStruct((NPAD, HH), jnp.float32),
                  jax.ShapeDtypeStruct((NPAD, HH), jnp.float32)),
        mesh=mesh,
        scratch_types=[
            pltpu.VMEM((DCH, CHUNK), jnp.int32),
            pltpu.VMEM((CHUNK, HH), jnp.float32),
            pltpu.VMEM_SHARED((NPAD, HH), jnp.float32),
            pltpu.SemaphoreType.DMA,
        ],
    )
    scat_k = pl.kernel(
        _scatter_body,
        out_type=(jax.ShapeDtypeStruct((NPAD, HH), jnp.float32),
                  jax.ShapeDtypeStruct((NPAD, HH), jnp.float32)),
        mesh=mesh,
        scratch_types=[
            pltpu.VMEM((2, CHUNK), jnp.int32),
            pltpu.VMEM((2, CHUNK), jnp.int32),
            pltpu.VMEM((2, CHUNK, HH), jnp.float32),
            pltpu.VMEM_SHARED((NPAD, HH), jnp.float32),
            pltpu.SemaphoreType.DMA,
            pltpu.SemaphoreType.DMA,
            pltpu.SemaphoreType.DMA,
            pltpu.SemaphoreType.DMA,
            pltpu.SemaphoreType.DMA,
            pltpu.SemaphoreType.DMA,
        ],
    )
    return deg_k, scat_k


def _mm0_body(x_ref, w_ref, o_ref):
    o_ref[...] = jnp.dot(x_ref[...], w_ref[...],
                         preferred_element_type=jnp.float32)


def _scale_body(deg0_ref, deg1_ref, hw_ref, o0_ref, o1_ref, dinv_ref):
    deg = deg0_ref[...][:N, 0:1] + deg1_ref[...][:N, 0:1]
    dinv = lax.rsqrt(deg + 1.0)
    u = hw_ref[...] * dinv
    o0_ref[...] = u[:, :HH]
    o1_ref[...] = u[:, HH:]
    dinv_ref[...] = jnp.broadcast_to(dinv, (N, 8))


def _epi_body(dinv_ref, a0_ref, a1_ref, u0_ref, u1_ref, b_ref, g_ref, be_ref,
              w_ref, o0_ref, o1_ref):
    dinv = dinv_ref[...][:, 0:1]
    acc = jnp.concatenate([a0_ref[...][:N], a1_ref[...][:N]], axis=1)
    u = jnp.concatenate([u0_ref[...], u1_ref[...]], axis=1)
    conv = dinv * (acc + u) + b_ref[...]
    h = jnp.maximum(conv, 0.0)
    m = jnp.mean(h, axis=0, keepdims=True)
    v = jnp.mean((h - m) * (h - m), axis=0, keepdims=True)
    hbn = g_ref[...] * (h - m) * lax.rsqrt(v + 1e-5) + be_ref[...]
    un = jnp.dot(hbn, w_ref[...], preferred_element_type=jnp.float32) * dinv
    o0_ref[...] = un[:, :HH]
    o1_ref[...] = un[:, HH:]


def _head_body(dinv_ref, a0_ref, a1_ref, u0_ref, u1_ref, b_ref, batch_ref,
               y_ref, muw_ref, mub_ref, sgw_ref, sgb_ref, f1w_ref, f1b_ref,
               f2w_ref, f2b_ref, logits_ref, loss_ref):
    dinv = dinv_ref[...][:, 0:1]
    acc = jnp.concatenate([a0_ref[...][:N], a1_ref[...][:N]], axis=1)
    u = jnp.concatenate([u0_ref[...], u1_ref[...]], axis=1)
    h = jnp.maximum(dinv * (acc + u) + b_ref[...], 0.0)
    gids = lax.broadcasted_iota(jnp.int32, (1, G), 1)
    p = (batch_ref[...] == gids).astype(jnp.float32)
    psum = lax.dot_general(p, h, (((0,), (0,)), ((), ())),
                           preferred_element_type=jnp.float32)
    ones = jnp.full((N, 1), 1.0, jnp.float32)
    cnt = lax.dot_general(p, ones, (((0,), (0,)), ((), ())),
                          preferred_element_type=jnp.float32)
    pooled = psum / jnp.maximum(cnt, 1.0)
    mu = jnp.dot(pooled, muw_ref[...], preferred_element_type=jnp.float32) \
        + mub_ref[...]
    sp = jnp.dot(pooled, sgw_ref[...], preferred_element_type=jnp.float32) \
        + sgb_ref[...]
    sigma = jnp.maximum(sp, 0.0) + jnp.log(1.0 + jnp.exp(-jnp.abs(sp)))
    ms = jnp.concatenate([mu, sigma], axis=1)
    hid = jnp.maximum(
        jnp.dot(ms, f1w_ref[...], preferred_element_type=jnp.float32)
        + f1b_ref[...], 0.0)
    logits = jnp.dot(hid, f2w_ref[...], preferred_element_type=jnp.float32) \
        + f2b_ref[...]
    mx = jnp.max(logits, axis=1, keepdims=True)
    sh = logits - mx
    lse = jnp.log(jnp.sum(jnp.exp(sh), axis=1, keepdims=True))
    logp = sh - lse
    cids = lax.broadcasted_iota(jnp.int32, (1, C), 1)
    picked = jnp.where(y_ref[...] == cids, logp, 0.0)
    loss = -jnp.sum(picked) / G
    logits_ref[...] = logits
    loss_ref[...] = jnp.reshape(loss, (1, 1))


def _tc_call(body, out_shapes):
    return pl.pallas_call(body, out_shape=out_shapes)


def kernel(x, edge_index, batch, y, W0, b0, g0, be0, W1, b1, g1, be1, W2, b2,
           muW, mub, sgW, sgb, f1W, f1b, f2W, f2b):
    src = edge_index[0].astype(jnp.int32)
    dst = edge_index[1].astype(jnp.int32)
    npad = EP - E
    pad_i = jnp.arange(npad, dtype=jnp.int32)
    src_p = jnp.concatenate([src, (pad_i * 37) % N]).reshape(EP // CHUNK, CHUNK)
    dst_p = jnp.concatenate([dst, N + (pad_i % NJUNK)]).reshape(
        EP // CHUNK, CHUNK)

    ones_h = jnp.ones((CHUNK, HH), jnp.float32)
    zero_h = jnp.zeros((RPT, HH), jnp.float32)

    deg_kernel, scatter_kernel = _sc_kernels()
    deg0, deg1 = deg_kernel(dst_p, ones_h, zero_h)

    hw0 = _tc_call(_mm0_body, jax.ShapeDtypeStruct((N, H), jnp.float32))(x, W0)
    uo = jax.ShapeDtypeStruct((N, HH), jnp.float32)
    dvo = jax.ShapeDtypeStruct((N, 8), jnp.float32)
    u0_lo, u0_hi, dinv = _tc_call(_scale_body, (uo, uo, dvo))(deg0, deg1, hw0)
    u0 = (u0_lo, u0_hi)

    a0 = scatter_kernel(u0[0], u0[1], src_p, dst_p, zero_h)
    u1 = _tc_call(_epi_body, (uo, uo))(
        dinv, a0[0], a0[1], u0[0], u0[1], b0.reshape(1, H), g0.reshape(1, H),
        be0.reshape(1, H), W1)

    a1 = scatter_kernel(u1[0], u1[1], src_p, dst_p, zero_h)
    u2 = _tc_call(_epi_body, (uo, uo))(
        dinv, a1[0], a1[1], u1[0], u1[1], b1.reshape(1, H), g1.reshape(1, H),
        be1.reshape(1, H), W2)

    a2 = scatter_kernel(u2[0], u2[1], src_p, dst_p, zero_h)
    logits, loss = _tc_call(
        _head_body, (jax.ShapeDtypeStruct((G, C), jnp.float32),
                     jax.ShapeDtypeStruct((1, 1), jnp.float32)))(
        dinv, a2[0], a2[1], u2[0], u2[1], b2.reshape(1, H),
        batch.astype(jnp.int32).reshape(N, 1), y.astype(jnp.int32).reshape(G, 1),
        muW, mub.reshape(1, H), sgW, sgb.reshape(1, H), f1W, f1b.reshape(1, H),
        f2W, f2b.reshape(1, C))
    return logits, loss.reshape(())


# speedup vs baseline: 1.2923x; 1.0027x over previous
import functools

import jax
import jax.numpy as jnp
from jax import lax
from jax.experimental import pallas as pl
from jax.experimental.pallas import tpu as pltpu
from jax.experimental.pallas import tpu_sc as plsc

N = 10000
E = 320000
D = 128
H = 256
HH = H // 2
G = 128
C = 16

NC = 2
NS = 16
CHUNK = 128
NCH = 160
EP = NCH * NS * CHUNK
EPT = EP // NS
EPW = EP // (NC * NS)
DCH = EPW // CHUNK
NJUNK = 16
NPAD = 10112
RPT = NPAD // NS


def _deg_body(dst2_hbm, ones_hbm, zero_hbm, out0, out1, didx2, ones_v, acc,
              sem):
    c = lax.axis_index("c")
    s = lax.axis_index("s")
    pltpu.sync_copy(zero_hbm, acc.at[pl.ds(s * RPT, RPT)])
    pltpu.sync_copy(ones_hbm, ones_v)
    w = c * NS + s
    pltpu.sync_copy(dst2_hbm.at[pl.ds(w * DCH, DCH)], didx2)
    plsc.subcore_barrier()

    def fire(j, carry):
        pltpu.make_async_copy(ones_v, acc.at[didx2.at[j]], sem).start(add=True)
        return carry

    def drain(j, carry):
        pltpu.make_async_copy(ones_v, acc.at[didx2.at[j]], sem).wait()
        return carry

    lax.fori_loop(0, DCH, fire, 0)
    lax.fori_loop(0, DCH, drain, 0)
    plsc.subcore_barrier()

    @pl.when(c == 0)
    def _():
        pltpu.sync_copy(acc.at[pl.ds(s * RPT, RPT)], out0.at[pl.ds(s * RPT, RPT)])

    @pl.when(c == 1)
    def _():
        pltpu.sync_copy(acc.at[pl.ds(s * RPT, RPT)], out1.at[pl.ds(s * RPT, RPT)])


def _scatter_body(u0_hbm, u1_hbm, src2_hbm, dst2_hbm, zero_hbm,
                  out0, out1, sidx, didx, rows, acc,
                  gsem0, gsem1, isem0, isem1, ssem0, ssem1):
    c = lax.axis_index("c")
    s = lax.axis_index("s")
    gsems = (gsem0, gsem1)
    isems = (isem0, isem1)
    ssems = (ssem0, ssem1)
    pltpu.sync_copy(zero_hbm, acc.at[pl.ds(s * RPT, RPT)])
    plsc.subcore_barrier()

    def idxload(j, b):
        pltpu.async_copy(src2_hbm.at[s * NCH + j], sidx.at[b], isems[b])
        pltpu.async_copy(dst2_hbm.at[s * NCH + j], didx.at[b], isems[b])

    def idxwait(b):
        pltpu.make_async_copy(src2_hbm.at[0], sidx.at[b], isems[b]).wait()
        pltpu.make_async_copy(dst2_hbm.at[0], didx.at[b], isems[b]).wait()

    def gather(b):
        @pl.when(c == 0)
        def _():
            pltpu.async_copy(u0_hbm.at[sidx.at[b]], rows.at[b], gsems[b])

        @pl.when(c == 1)
        def _():
            pltpu.async_copy(u1_hbm.at[sidx.at[b]], rows.at[b], gsems[b])

    def gwait(b):
        pltpu.make_async_copy(u0_hbm.at[sidx.at[b]], rows.at[b],
                              gsems[b]).wait()

    idxload(0, 0)
    idxload(1, 1)
    idxwait(0)
    gather(0)

    def body(j2, carry):
        for b in range(2):
            j = j2 * 2 + b

            @pl.when(j + 1 < NCH)
            def _():
                idxwait(1 - b)
                gather(1 - b)

            gwait(b)
            pltpu.sync_copy(rows.at[b], acc.at[didx.at[b]], add=True)

            @pl.when(j + 2 < NCH)
            def _():
                idxload(j + 2, b)
        return carry

    lax.fori_loop(0, NCH // 2, body, 0)
    plsc.subcore_barrier()

    @pl.when(c == 0)
    def _():
        pltpu.sync_copy(acc.at[pl.ds(s * RPT, RPT)], out0.at[pl.ds(s * RPT, RPT)])

    @pl.when(c == 1)
    def _():
        pltpu.sync_copy(acc.at[pl.ds(s * RPT, RPT)], out1.at[pl.ds(s * RPT, RPT)])


@functools.lru_cache(maxsize=None)
def _sc_kernels():
    mesh = plsc.VectorSubcoreMesh(
        core_axis_name="c", subcore_axis_name="s",
        num_cores=NC, num_subcores=NS)
    deg_k = pl.kernel(
        _deg_body,
        out_type=(jax.ShapeDtypeStruct((NPAD, HH), jnp.float32),
                  jax.ShapeDtypeStruct((NPAD, HH), jnp.float32)),
        mesh=mesh,
        scratch_types=[
            pltpu.VMEM((DCH, CHUNK), jnp.int32),
            pltpu.VMEM((CHUNK, HH), jnp.float32),
            pltpu.VMEM_SHARED((NPAD, HH), jnp.float32),
            pltpu.SemaphoreType.DMA,
        ],
    )
    scat_k = pl.kernel(
        _scatter_body,
        out_type=(jax.ShapeDtypeStruct((NPAD, HH), jnp.float32),
                  jax.ShapeDtypeStruct((NPAD, HH), jnp.float32)),
        mesh=mesh,
        scratch_types=[
            pltpu.VMEM((2, CHUNK), jnp.int32),
            pltpu.VMEM((2, CHUNK), jnp.int32),
            pltpu.VMEM((2, CHUNK, HH), jnp.float32),
            pltpu.VMEM_SHARED((NPAD, HH), jnp.float32),
            pltpu.SemaphoreType.DMA,
            pltpu.SemaphoreType.DMA,
            pltpu.SemaphoreType.DMA,
            pltpu.SemaphoreType.DMA,
            pltpu.SemaphoreType.DMA,
            pltpu.SemaphoreType.DMA,
        ],
    )
    return deg_k, scat_k


def _mm0_body(x_ref, w_ref, o_ref):
    o_ref[...] = jnp.dot(x_ref[...], w_ref[...],
                         preferred_element_type=jnp.float32)


def _scale_body(deg0_ref, deg1_ref, hw_ref, o0_ref, o1_ref, dinv_ref):
    deg = deg0_ref[...][:N, 0:1] + deg1_ref[...][:N, 0:1]
    dinv = lax.rsqrt(deg + 1.0)
    u = hw_ref[...] * dinv
    o0_ref[...] = u[:, :HH]
    o1_ref[...] = u[:, HH:]
    dinv_ref[...] = jnp.broadcast_to(dinv, (N, 8))


def _epi_body(dinv_ref, a0_ref, a1_ref, u0_ref, u1_ref, b_ref, g_ref, be_ref,
              w_ref, o0_ref, o1_ref):
    dinv = dinv_ref[...][:, 0:1]
    acc = jnp.concatenate([a0_ref[...][:N], a1_ref[...][:N]], axis=1)
    u = jnp.concatenate([u0_ref[...], u1_ref[...]], axis=1)
    conv = dinv * (acc + u) + b_ref[...]
    h = jnp.maximum(conv, 0.0)
    m = jnp.mean(h, axis=0, keepdims=True)
    v = jnp.mean((h - m) * (h - m), axis=0, keepdims=True)
    hbn = g_ref[...] * (h - m) * lax.rsqrt(v + 1e-5) + be_ref[...]
    un = jnp.dot(hbn, w_ref[...], preferred_element_type=jnp.float32) * dinv
    o0_ref[...] = un[:, :HH]
    o1_ref[...] = un[:, HH:]


def _head_body(dinv_ref, a0_ref, a1_ref, u0_ref, u1_ref, b_ref, batch_ref,
               y_ref, muw_ref, mub_ref, sgw_ref, sgb_ref, f1w_ref, f1b_ref,
               f2w_ref, f2b_ref, logits_ref, loss_ref):
    dinv = dinv_ref[...][:, 0:1]
    acc = jnp.concatenate([a0_ref[...][:N], a1_ref[...][:N]], axis=1)
    u = jnp.concatenate([u0_ref[...], u1_ref[...]], axis=1)
    h = jnp.maximum(dinv * (acc + u) + b_ref[...], 0.0)
    gids = lax.broadcasted_iota(jnp.int32, (1, G), 1)
    p = (batch_ref[...] == gids).astype(jnp.float32)
    psum = lax.dot_general(p, h, (((0,), (0,)), ((), ())),
                           preferred_element_type=jnp.float32)
    ones = jnp.full((N, 1), 1.0, jnp.float32)
    cnt = lax.dot_general(p, ones, (((0,), (0,)), ((), ())),
                          preferred_element_type=jnp.float32)
    pooled = psum / jnp.maximum(cnt, 1.0)
    mu = jnp.dot(pooled, muw_ref[...], preferred_element_type=jnp.float32) \
        + mub_ref[...]
    sp = jnp.dot(pooled, sgw_ref[...], preferred_element_type=jnp.float32) \
        + sgb_ref[...]
    sigma = jnp.maximum(sp, 0.0) + jnp.log(1.0 + jnp.exp(-jnp.abs(sp)))
    ms = jnp.concatenate([mu, sigma], axis=1)
    hid = jnp.maximum(
        jnp.dot(ms, f1w_ref[...], preferred_element_type=jnp.float32)
        + f1b_ref[...], 0.0)
    logits = jnp.dot(hid, f2w_ref[...], preferred_element_type=jnp.float32) \
        + f2b_ref[...]
    mx = jnp.max(logits, axis=1, keepdims=True)
    sh = logits - mx
    lse = jnp.log(jnp.sum(jnp.exp(sh), axis=1, keepdims=True))
    logp = sh - lse
    cids = lax.broadcasted_iota(jnp.int32, (1, C), 1)
    picked = jnp.where(y_ref[...] == cids, logp, 0.0)
    loss = -jnp.sum(picked) / G
    logits_ref[...] = logits
    loss_ref[...] = jnp.reshape(loss, (1, 1))


def _tc_call(body, out_shapes):
    return pl.pallas_call(body, out_shape=out_shapes)


def kernel(x, edge_index, batch, y, W0, b0, g0, be0, W1, b1, g1, be1, W2, b2,
           muW, mub, sgW, sgb, f1W, f1b, f2W, f2b):
    src = edge_index[0].astype(jnp.int32)
    dst = edge_index[1].astype(jnp.int32)
    npad = EP - E
    pad_i = jnp.arange(npad, dtype=jnp.int32)
    src_p = jnp.concatenate([src, (pad_i * 37) % N]).reshape(EP // CHUNK, CHUNK)
    dst_p = jnp.concatenate([dst, N + (pad_i % NJUNK)]).reshape(
        EP // CHUNK, CHUNK)

    ones_h = jnp.ones((CHUNK, HH), jnp.float32)
    zero_h = jnp.zeros((RPT, HH), jnp.float32)

    deg_kernel, scatter_kernel = _sc_kernels()
    deg0, deg1 = deg_kernel(dst_p, ones_h, zero_h)

    hw0 = _tc_call(_mm0_body, jax.ShapeDtypeStruct((N, H), jnp.float32))(x, W0)
    uo = jax.ShapeDtypeStruct((N, HH), jnp.float32)
    dvo = jax.ShapeDtypeStruct((N, 8), jnp.float32)
    u0_lo, u0_hi, dinv = _tc_call(_scale_body, (uo, uo, dvo))(deg0, deg1, hw0)
    u0 = (u0_lo, u0_hi)

    a0 = scatter_kernel(u0[0], u0[1], src_p, dst_p, zero_h)
    u1 = _tc_call(_epi_body, (uo, uo))(
        dinv, a0[0], a0[1], u0[0], u0[1], b0.reshape(1, H), g0.reshape(1, H),
        be0.reshape(1, H), W1)

    a1 = scatter_kernel(u1[0], u1[1], src_p, dst_p, zero_h)
    u2 = _tc_call(_epi_body, (uo, uo))(
        dinv, a1[0], a1[1], u1[0], u1[1], b1.reshape(1, H), g1.reshape(1, H),
        be1.reshape(1, H), W2)

    a2 = scatter_kernel(u2[0], u2[1], src_p, dst_p, zero_h)
    logits, loss = _tc_call(
        _head_body, (jax.ShapeDtypeStruct((G, C), jnp.float32),
                     jax.ShapeDtypeStruct((1, 1), jnp.float32)))(
        dinv, a2[0], a2[1], u2[0], u2[1], b2.reshape(1, H),
        batch.astype(jnp.int32).reshape(N, 1), y.astype(jnp.int32).reshape(G, 1),
        muW, mub.reshape(1, H), sgW, sgb.reshape(1, H), f1W, f1b.reshape(1, H),
        f2W, f2b.reshape(1, C))
    return logits, loss.reshape(())
